# baseline (speedup 1.0000x reference)
import functools

import jax
import jax.numpy as jnp
from jax import lax
from jax.experimental import pallas as pl
from jax.experimental.pallas import tpu as pltpu
from jax.experimental.pallas import tpu_sc as plsc

HID = 64
MAXN = 4
MAXL = 4
CUT = 5.0
CUT3 = 4.0
NLAYERS = 4
N = 10000
E = 160000
T = 160000

NC = 2
NS = 16
NW = NC * NS
CH = 128

NPAD = 10240
EPAD = 172032
NB = 1024
EB = 4096

RACC = 16368
ACCROWS = RACC + 16
NACCROWS = NPAD + 16

_f32 = jnp.float32


def _sigmoid(x):
    return jax.nn.sigmoid(x)


def _swish(x):
    return x * jax.nn.sigmoid(x)


def _bessel(r, c):
    rc = jnp.clip(r, 1e-6, None)
    out = []
    s = jnp.sqrt(2.0 / c)
    for n in range(1, MAXN + 1):
        out.append(s * jnp.sin((n * jnp.pi / c) * rc) / rc)
    return jnp.stack(out, axis=-1)


def _cutoff(r, c):
    x = jnp.clip(r / c, 0.0, 1.0)
    return 1.0 - 6.0 * x ** 5 + 15.0 * x ** 4 - 10.0 * x ** 3


def _sc_mesh():
    return plsc.VectorSubcoreMesh(core_axis_name="c", subcore_axis_name="s")


NG = 6
GCH = NG * CH


def _sc_gather(table, idxs, depth):
    n = len(idxs)
    b = idxs[0].shape[0]
    per_w = b // NW
    nch = per_w // CH
    gr = nch // NG

    @functools.partial(
        pl.kernel,
        out_type=[jax.ShapeDtypeStruct((b, depth), _f32) for _ in range(n)],
        mesh=_sc_mesh(),
        compiler_params=pltpu.CompilerParams(use_tc_tiling_on_sc=False),
        scratch_types=[
            pltpu.VMEM((per_w,), jnp.int32),
            pltpu.VMEM((2, GCH, depth), _f32),
            pltpu.SemaphoreType.DMA,
            pltpu.SemaphoreType.DMA,
        ],
    )
    def k(table_ref, *rest):
        idx_refs = rest[:n]
        out_refs = rest[n:2 * n]
        idx_s, rows3, gsem, wsem = rest[2 * n:]
        wid = lax.axis_index("s") * NC + lax.axis_index("c")
        base = wid * per_w
        for j in range(n):
            pltpu.sync_copy(idx_refs[j].at[pl.ds(base, per_w)], idx_s)

            def body(g, carry):
                r = lax.rem(g, 2)
                o = 1 - r

                @pl.when(g < gr)
                def _():
                    @pl.when(g >= 2)
                    def _():
                        pltpu.make_async_copy(
                            table_ref.at[pl.ds(0, GCH)], rows3.at[r], wsem).wait()
                    for bb in range(NG):
                        pltpu.async_copy(
                            table_ref.at[idx_s.at[pl.ds(g * GCH + bb * CH, CH)]],
                            rows3.at[r, pl.ds(bb * CH, CH)], gsem)

                @pl.when(g > 0)
                def _():
                    pltpu.make_async_copy(
                        table_ref.at[pl.ds(0, GCH)], rows3.at[o], gsem).wait()
                    pltpu.async_copy(
                        rows3.at[o],
                        out_refs[j].at[pl.ds(base + (g - 1) * GCH, GCH)], wsem)
                return carry

            lax.fori_loop(0, gr + 1, body, 0)
            pltpu.make_async_copy(table_ref.at[pl.ds(0, GCH)], rows3.at[0], wsem).wait()
            pltpu.make_async_copy(table_ref.at[pl.ds(0, GCH)], rows3.at[0], wsem).wait()

    res = k(table, *idxs)
    return list(res) if isinstance(res, (list, tuple)) else [res]


def _fill_zero(zero_s, rows):
    def zbody(i, carry):
        for j in range(HID // 16):
            zero_s[i, pl.ds(j * 16, 16)] = jnp.zeros((16,), _f32)
        return carry
    lax.fori_loop(0, rows, zbody, 0)


def _scatter_pipeline(rows_ref, idx_ref, acc, rows3, idx2, idxb,
                      rsem, isem, asem, base, gr, lo, bound, dump, ng):
    gch = ng * CH
    def body(g, carry):
        r = lax.rem(g, 2)
        o = 1 - r

        @pl.when(g < gr)
        def _():
            @pl.when(g >= 2)
            def _():
                pltpu.make_async_copy(
                    rows_ref.at[pl.ds(0, gch)], rows3.at[r], asem).wait()
            off = base + g * gch
            pltpu.async_copy(rows_ref.at[pl.ds(off, gch)], rows3.at[r], rsem)
            pltpu.async_copy(idx_ref.at[pl.ds(off, gch)], idx2.at[r], isem)

        @pl.when(g > 0)
        def _():
            pltpu.make_async_copy(rows_ref.at[pl.ds(0, gch)], rows3.at[o], rsem).wait()
            pltpu.make_async_copy(idx_ref.at[pl.ds(0, gch)], idx2.at[o], isem).wait()
            for bb in range(ng):
                row = o * ng + bb
                for j in range(CH // 16):
                    v = idx2[o, pl.ds(bb * CH + j * 16, 16)]
                    lv = v - lo
                    ok = (lv >= 0) & (lv < bound)
                    idxb[row, pl.ds(j * 16, 16)] = jnp.where(ok, lv, dump)
                pltpu.async_copy(rows3.at[o, pl.ds(bb * CH, CH)],
                                 acc.at[idxb.at[row]], asem, add=True)
        return carry

    lax.fori_loop(0, gr + 1, body, 0)
    pltpu.make_async_copy(rows_ref.at[pl.ds(0, gch)], rows3.at[0], asem).wait()
    pltpu.make_async_copy(rows_ref.at[pl.ds(0, gch)], rows3.at[0], asem).wait()


NGS = 3
GCHS = NGS * CH

_SC_SCR = [
    pltpu.VMEM((2, GCHS, HID), _f32),
    pltpu.VMEM((2, GCHS), jnp.int32),
    pltpu.VMEM((2 * NGS, CH), jnp.int32),
    pltpu.VMEM((128, HID), _f32),
    pltpu.SemaphoreType.DMA,
    pltpu.SemaphoreType.DMA,
    pltpu.SemaphoreType.DMA,
]


def _sc_scatter_n(rows, idx):
    per_w = EPAD // NW
    gr = per_w // GCHS
    tz = NACCROWS // NS
    wb = NPAD // NS

    @functools.partial(
        pl.kernel,
        out_type=jax.ShapeDtypeStruct((NC, NPAD, HID), _f32),
        mesh=_sc_mesh(),
        compiler_params=pltpu.CompilerParams(use_tc_tiling_on_sc=False),
        scratch_types=_SC_SCR + [pltpu.VMEM_SHARED((NACCROWS, HID), _f32)],
    )
    def k(rows_ref, idx_ref, out_ref, rows3, idx2, idxb, zero_s,
          rsem, isem, asem, acc):
        c = lax.axis_index("c")
        s = lax.axis_index("s")
        _fill_zero(zero_s, 128)
        zoff = s * tz
        for sz, o in ((128, 0), (128, 128), (128, 256), (128, 384), (tz - 512, 512)):
            pltpu.sync_copy(zero_s.at[pl.ds(0, sz)], acc.at[pl.ds(zoff + o, sz)])
        plsc.subcore_barrier()
        base = (s * NC + c) * per_w
        _scatter_pipeline(rows_ref, idx_ref, acc, rows3, idx2, idxb,
                          rsem, isem, asem, base, gr, 0, NPAD, NPAD, NGS)
        plsc.subcore_barrier()
        pltpu.sync_copy(acc.at[pl.ds(s * wb, wb)], out_ref.at[c, pl.ds(s * wb, wb)])

    return k(rows, idx)


def _sc_scatter_e(rows, idx):
    per_tile = EPAD // NS
    gr = per_tile // GCHS
    tz = ACCROWS // NS
    wb = RACC // NS

    @functools.partial(
        pl.kernel,
        out_type=jax.ShapeDtypeStruct((EPAD, HID), _f32),
        mesh=_sc_mesh(),
        compiler_params=pltpu.CompilerParams(use_tc_tiling_on_sc=False),
        scratch_types=_SC_SCR + [pltpu.VMEM_SHARED((ACCROWS, HID), _f32)],
    )
    def k(rows_ref, idx_ref, out_ref, rows3, idx2, idxb, zero_s,
          rsem, isem, asem, acc):
        c = lax.axis_index("c")
        s = lax.axis_index("s")
        _fill_zero(zero_s, 128)
        for p in range(5):
            lo = (2 * p + c) * RACC
            for q in range(tz // 128):
                pltpu.sync_copy(zero_s, acc.at[pl.ds(s * tz + q * 128, 128)])
            plsc.subcore_barrier()
            _scatter_pipeline(rows_ref, idx_ref, acc, rows3, idx2, idxb,
                              rsem, isem, asem, s * per_tile, gr, lo, RACC, RACC, NGS)
            plsc.subcore_barrier()
            pltpu.sync_copy(acc.at[pl.ds(s * wb, wb)], out_ref.at[pl.ds(lo + s * wb, wb)])
            plsc.subcore_barrier()

    return k(rows, idx)


def _k1_body(z_ref, wemb_ref, sc_ref, sh_ref, atoms_ref, scz_ref, shz_ref):
    z = z_ref[:, 0]
    oh = (jax.lax.broadcasted_iota(jnp.int32, (NB, 128), 1) == z[:, None]).astype(_f32)
    atoms_ref[...] = jnp.dot(oh, wemb_ref[...], preferred_element_type=_f32)
    scz_ref[...] = jnp.dot(oh, sc_ref[...], preferred_element_type=_f32)
    shz_ref[...] = jnp.dot(oh, sh_ref[...], preferred_element_type=_f32)


def _k1(z_pad, wemb_pad, scale_pad, shift_pad):
    return pl.pallas_call(
        _k1_body,
        grid=(NPAD // NB,),
        in_specs=[
            pl.BlockSpec((NB, 1), lambda i: (i, 0)),
            pl.BlockSpec((128, HID), lambda i: (0, 0)),
            pl.BlockSpec((128, 1), lambda i: (0, 0)),
            pl.BlockSpec((128, 1), lambda i: (0, 0)),
        ],
        out_specs=[
            pl.BlockSpec((NB, HID), lambda i: (i, 0)),
            pl.BlockSpec((NB, 1), lambda i: (i, 0)),
            pl.BlockSpec((NB, 1), lambda i: (i, 0)),
        ],
        out_shape=[
            jax.ShapeDtypeStruct((NPAD, HID), _f32),
            jax.ShapeDtypeStruct((NPAD, 1), _f32),
            jax.ShapeDtypeStruct((NPAD, 1), _f32),
        ],
    )(z_pad, wemb_pad, scale_pad, shift_pad)


def _k2_body(g0_ref, g1_ref, pbc_ref, cell_ref, wedge_ref, geom_ref, eattr_ref):
    off = jnp.dot(pbc_ref[...], cell_ref[...], preferred_element_type=_f32)
    d = g0_ref[...] - g1_ref[...] - off
    elen = jnp.sqrt(jnp.sum(d * d, axis=1))
    ez = _bessel(elen, CUT)
    c3 = _cutoff(elen, CUT3)
    geom_ref[...] = jnp.concatenate(
        [d[:, :3], elen[:, None], ez, c3[:, None], jnp.zeros((EB, 7), _f32)], axis=1)
    eattr_ref[...] = jnp.dot(ez, wedge_ref[...], preferred_element_type=_f32)


def _k2(g0, g1, pbc_pad, cell_pad, wedge):
    return pl.pallas_call(
        _k2_body,
        grid=(EPAD // EB,),
        in_specs=[
            pl.BlockSpec((EB, 16), lambda i: (i, 0)),
            pl.BlockSpec((EB, 16), lambda i: (i, 0)),
            pl.BlockSpec((EB, 16), lambda i: (i, 0)),
            pl.BlockSpec((16, 16), lambda i: (0, 0)),
            pl.BlockSpec((MAXN, HID), lambda i: (0, 0)),
        ],
        out_specs=[
            pl.BlockSpec((EB, 16), lambda i: (i, 0)),
            pl.BlockSpec((EB, HID), lambda i: (i, 0)),
        ],
        out_shape=[
            jax.ShapeDtypeStruct((EPAD, 16), _f32),
            jax.ShapeDtypeStruct((EPAD, HID), _f32),
        ],
    )(g0, g1, pbc_pad, cell_pad, wedge)


def _k3_body(gj_ref, gk_ref, out_ref):
    gj = gj_ref[...]
    gk = gk_ref[...]
    rij = gj[:, 3]
    rik = gk[:, 3]
    dot = jnp.sum(gj[:, :3] * gk[:, :3], axis=1)
    cos = dot / jnp.clip(rij * rik, 1e-8, None)
    cos = jnp.clip(cos, -1.0 + 1e-7, 1.0 - 1e-7)
    leg = jnp.stack(
        [jnp.ones_like(cos), cos, 0.5 * (3.0 * cos ** 2 - 1.0),
         0.5 * (5.0 * cos ** 3 - 3.0 * cos)], axis=-1)
    rad = gk[:, 4:8]
    three = (rad[:, None, :] * leg[:, :, None]).reshape(EB, MAXL * MAXN)
    w3 = gj[:, 8] * gk[:, 8]
    out_ref[...] = three * w3[:, None]


def _k3(gj, gk):
    return pl.pallas_call(
        _k3_body,
        grid=(EPAD // EB,),
        in_specs=[
            pl.BlockSpec((EB, 16), lambda i: (i, 0)),
            pl.BlockSpec((EB, 16), lambda i: (i, 0)),
        ],
        out_specs=pl.BlockSpec((EB, 16), lambda i: (i, 0)),
        out_shape=jax.ShapeDtypeStruct((EPAD, 16), _f32),
    )(gj, gk)


def _k4_body(tw_ref, ak_ref, wsbf_ref, wk_ref, m3_ref):
    a = jnp.dot(tw_ref[...], wsbf_ref[...], preferred_element_type=_f32)
    b = _sigmoid(jnp.dot(ak_ref[...], wk_ref[...], preferred_element_type=_f32))
    m3_ref[...] = a * b


def _k4(three_w, ak, wsbf_l, wk_l):
    return pl.pallas_call(
        _k4_body,
        grid=(EPAD // EB,),
        in_specs=[
            pl.BlockSpec((EB, 16), lambda i: (i, 0)),
            pl.BlockSpec((EB, HID), lambda i: (i, 0)),
            pl.BlockSpec((16, HID), lambda i: (0, 0)),
            pl.BlockSpec((HID, HID), lambda i: (0, 0)),
        ],
        out_specs=pl.BlockSpec((EB, HID), lambda i: (i, 0)),
        out_shape=jax.ShapeDtypeStruct((EPAD, HID), _f32),
    )(three_w, ak, wsbf_l, wk_l)


def _k6_body(a0_ref, a1_ref, eattr_ref, agg_ref, geom_ref,
             wg1_ref, wg2_ref, we_ref, wa_ref, wer_ref, war_ref,
             eout_ref, msg_ref):
    agg = agg_ref[...]
    eattr = eattr_ref[...]
    a0 = a0_ref[...]
    a1 = a1_ref[...]
    geom = geom_ref[...]

    g1 = jnp.dot(agg, wg1_ref[...], preferred_element_type=_f32)
    g2 = jnp.dot(agg, wg2_ref[...], preferred_element_type=_f32)
    eattr = eattr + _swish(g1) * _sigmoid(g2)

    er = jnp.dot(geom, wer_ref[...], preferred_element_type=_f32)
    ar = jnp.dot(geom, war_ref[...], preferred_element_type=_f32)

    a01 = jnp.concatenate([a0, a1], axis=1)
    we = we_ref[...]
    wa = wa_ref[...]
    p = jnp.dot(a01, we[:128], preferred_element_type=_f32)
    e1 = p[:, :HID] + jnp.dot(eattr, we[128:192, :HID], preferred_element_type=_f32)
    e2 = p[:, HID:] + jnp.dot(eattr, we[128:192, HID:], preferred_element_type=_f32)
    eattr = eattr + _swish(e1) * _sigmoid(e2) * er
    eout_ref[...] = eattr

    q = jnp.dot(a01, wa[:128], preferred_element_type=_f32)
    m1 = q[:, :HID] + jnp.dot(eattr, wa[128:192, :HID], preferred_element_type=_f32)
    m2 = q[:, HID:] + jnp.dot(eattr, wa[128:192, HID:], preferred_element_type=_f32)
    msg_ref[...] = _swish(m1) * _sigmoid(m2) * ar


def _k6(a0, a1, eattr, agg, geom, wg1_l, wg2_l, we_l, wa_l, wer_l, war_l):
    return pl.pallas_call(
        _k6_body,
        grid=(EPAD // EB,),
        in_specs=[
            pl.BlockSpec((EB, HID), lambda i: (i, 0)),
            pl.BlockSpec((EB, HID), lambda i: (i, 0)),
            pl.BlockSpec((EB, HID), lambda i: (i, 0)),
            pl.BlockSpec((EB, HID), lambda i: (i, 0)),
            pl.BlockSpec((EB, 16), lambda i: (i, 0)),
            pl.BlockSpec((HID, HID), lambda i: (0, 0)),
            pl.BlockSpec((HID, HID), lambda i: (0, 0)),
            pl.BlockSpec((192, 128), lambda i: (0, 0)),
            pl.BlockSpec((192, 128), lambda i: (0, 0)),
            pl.BlockSpec((16, HID), lambda i: (0, 0)),
            pl.BlockSpec((16, HID), lambda i: (0, 0)),
        ],
        out_specs=[
            pl.BlockSpec((EB, HID), lambda i: (i, 0)),
            pl.BlockSpec((EB, HID), lambda i: (i, 0)),
        ],
        out_shape=[
            jax.ShapeDtypeStruct((EPAD, HID), _f32),
            jax.ShapeDtypeStruct((EPAD, HID), _f32),
        ],
    )(a0, a1, eattr, agg, geom, wg1_l, wg2_l, we_l, wa_l, wer_l, war_l)


def _k8_body(a_ref, p_ref, out_ref):
    out_ref[...] = a_ref[...] + p_ref[0] + p_ref[1]


def _k8(atoms, part):
    return pl.pallas_call(
        _k8_body,
        grid=(NPAD // NB,),
        in_specs=[
            pl.BlockSpec((NB, HID), lambda i: (i, 0)),
            pl.BlockSpec((NC, NB, HID), lambda i: (0, i, 0)),
        ],
        out_specs=pl.BlockSpec((NB, HID), lambda i: (i, 0)),
        out_shape=jax.ShapeDtypeStruct((NPAD, HID), _f32),
    )(atoms, part)


def _k9_body(atoms_ref, scz_ref, shz_ref, f1_ref, f2_ref, f3_ref,
             v1_ref, v2_ref, v3_ref, out_ref):
    a = atoms_ref[...]
    h = _swish(jnp.dot(a, f1_ref[...], preferred_element_type=_f32))
    h = _swish(jnp.dot(h, f2_ref[...], preferred_element_type=_f32))
    h = jnp.dot(h, f3_ref[...], preferred_element_type=_f32)
    g = _swish(jnp.dot(a, v1_ref[...], preferred_element_type=_f32))
    g = _swish(jnp.dot(g, v2_ref[...], preferred_element_type=_f32))
    g = _sigmoid(jnp.dot(g, v3_ref[...], preferred_element_type=_f32))
    e = h[:, 0] * g[:, 0] * scz_ref[:, 0] + shz_ref[:, 0]

    @pl.when(pl.program_id(0) == 0)
    def _():
        out_ref[...] = jnp.zeros_like(out_ref)

    out_ref[...] += jnp.sum(e)[None, None]


def _k9(atoms, scz, shz, F1, F2, F3, V1, V2, V3):
    return pl.pallas_call(
        _k9_body,
        grid=(NPAD // NB,),
        in_specs=[
            pl.BlockSpec((NB, HID), lambda i: (i, 0)),
            pl.BlockSpec((NB, 1), lambda i: (i, 0)),
            pl.BlockSpec((NB, 1), lambda i: (i, 0)),
            pl.BlockSpec((HID, HID), lambda i: (0, 0)),
            pl.BlockSpec((HID, HID), lambda i: (0, 0)),
            pl.BlockSpec((HID, 1), lambda i: (0, 0)),
            pl.BlockSpec((HID, HID), lambda i: (0, 0)),
            pl.BlockSpec((HID, HID), lambda i: (0, 0)),
            pl.BlockSpec((HID, 1), lambda i: (0, 0)),
        ],
        out_specs=pl.BlockSpec((1, 1), lambda i: (0, 0)),
        out_shape=jax.ShapeDtypeStruct((1, 1), _f32),
    )(atoms, scz, shz, F1, F2, F3, V1, V2, V3)


def _pad1i(x, total, val):
    return jnp.full((total,), val, jnp.int32).at[:x.shape[0]].set(x.astype(jnp.int32))


def kernel(atom_pos, cell, pbc_offsets, atom_attr, edge_index,
           three_body_indices, num_three_body, num_triple_ij, num_atoms,
           num_bonds, num_graphs, W_embed, W_edge, Wsbf, Wk, Wg1, Wg2,
           We1, We2, Wer, Wa1, Wa2, War, F1, F2, F3, V1, V2, V3,
           scale, shift):
    i0 = edge_index[0]
    i1 = edge_index[1]
    t0 = three_body_indices[:, 0]
    t1 = three_body_indices[:, 1]

    i0g = _pad1i(i0, EPAD, 0)
    i1g = _pad1i(i1, EPAD, 0)
    t0g = _pad1i(t0, EPAD, 0)
    t1g = _pad1i(t1, EPAD, 0)
    i0s = _pad1i(i0, EPAD, NPAD)
    t0s = _pad1i(t0, EPAD, 2000000000)

    z_pad = jnp.full((NPAD, 1), 127, jnp.int32).at[:N, 0].set(atom_attr[:, 0].astype(jnp.int32))
    wemb_pad = jnp.zeros((128, HID), _f32).at[:95].set(W_embed)
    scale_pad = jnp.zeros((128, 1), _f32).at[:95, 0].set(scale)
    shift_pad = jnp.zeros((128, 1), _f32).at[:95, 0].set(shift)
    pos_pad = jnp.zeros((N, 16), _f32).at[:, :3].set(atom_pos)
    pbc_pad = jnp.zeros((EPAD, 16), _f32).at[:E, :3].set(pbc_offsets)
    cell_pad = jnp.zeros((16, 16), _f32).at[:3, :3].set(cell[0])
    we_pack = jnp.concatenate([We1, We2], axis=2)
    wa_pack = jnp.concatenate([Wa1, Wa2], axis=2)
    wer_pad = jnp.zeros((NLAYERS, 16, HID), _f32).at[:, 4:8].set(Wer)
    war_pad = jnp.zeros((NLAYERS, 16, HID), _f32).at[:, 4:8].set(War)

    atoms, scz, shz = _k1(z_pad, wemb_pad, scale_pad, shift_pad)

    g0, g1 = _sc_gather(pos_pad, [i0g, i1g], 16)
    geom, eattr = _k2(g0, g1, pbc_pad, cell_pad, W_edge)

    gj, gk = _sc_gather(geom, [t0g, t1g], 16)
    three_w = _k3(gj, gk)

    for l in range(NLAYERS):
        a0, a1 = _sc_gather(atoms, [i0g, i1g], HID)
        (ak,) = _sc_gather(a1, [t1g], HID)
        m3 = _k4(three_w, ak, Wsbf[l], Wk[l])
        agg = _sc_scatter_e(m3, t0s)
        eattr, msg = _k6(a0, a1, eattr, agg, geom,
                         Wg1[l], Wg2[l], we_pack[l], wa_pack[l],
                         wer_pad[l], war_pad[l])
        part = _sc_scatter_n(msg, i0s)
        atoms = _k8(atoms, part)

    out = _k9(atoms, scz, shz, F1, F2, F3, V1, V2, V3)
    return out[0]


# speedup vs baseline: 1.1615x; 1.1615x over previous
import functools

import jax
import jax.numpy as jnp
from jax import lax
from jax.experimental import pallas as pl
from jax.experimental.pallas import tpu as pltpu
from jax.experimental.pallas import tpu_sc as plsc

HID = 64
MAXN = 4
MAXL = 4
CUT = 5.0
CUT3 = 4.0
NLAYERS = 4
N = 10000
E = 160000
T = 160000

NC = 2
NS = 16
NW = NC * NS
CH = 128

NPAD = 10240
EPAD = 172032
NB = 1024
EB = 4096

RACC = 20464
ACCROWS = RACC + 16
NACCROWS = NPAD + 16

_f32 = jnp.float32


def _sigmoid(x):
    return jax.nn.sigmoid(x)


def _swish(x):
    return x * jax.nn.sigmoid(x)


def _bessel(r, c):
    rc = jnp.clip(r, 1e-6, None)
    out = []
    s = jnp.sqrt(2.0 / c)
    for n in range(1, MAXN + 1):
        out.append(s * jnp.sin((n * jnp.pi / c) * rc) / rc)
    return jnp.stack(out, axis=-1)


def _cutoff(r, c):
    x = jnp.clip(r / c, 0.0, 1.0)
    return 1.0 - 6.0 * x ** 5 + 15.0 * x ** 4 - 10.0 * x ** 3


def _sc_mesh():
    return plsc.VectorSubcoreMesh(core_axis_name="c", subcore_axis_name="s")


NG = 6
GCH = NG * CH


def _sc_gather(table, idxs, depth):
    n = len(idxs)
    b = idxs[0].shape[0]
    per_w = b // NW
    nch = per_w // CH
    gr = nch // NG

    @functools.partial(
        pl.kernel,
        out_type=[jax.ShapeDtypeStruct((b, depth), _f32) for _ in range(n)],
        mesh=_sc_mesh(),
        compiler_params=pltpu.CompilerParams(use_tc_tiling_on_sc=False),
        scratch_types=[
            pltpu.VMEM((per_w,), jnp.int32),
            pltpu.VMEM((2, GCH, depth), _f32),
            pltpu.SemaphoreType.DMA,
            pltpu.SemaphoreType.DMA,
        ],
    )
    def k(table_ref, *rest):
        idx_refs = rest[:n]
        out_refs = rest[n:2 * n]
        idx_s, rows3, gsem, wsem = rest[2 * n:]
        wid = lax.axis_index("s") * NC + lax.axis_index("c")
        base = wid * per_w
        for j in range(n):
            pltpu.sync_copy(idx_refs[j].at[pl.ds(base, per_w)], idx_s)

            def body(g, carry):
                r = lax.rem(g, 2)
                o = 1 - r

                @pl.when(g < gr)
                def _():
                    @pl.when(g >= 2)
                    def _():
                        pltpu.make_async_copy(
                            table_ref.at[pl.ds(0, GCH)], rows3.at[r], wsem).wait()
                    for bb in range(NG):
                        pltpu.async_copy(
                            table_ref.at[idx_s.at[pl.ds(g * GCH + bb * CH, CH)]],
                            rows3.at[r, pl.ds(bb * CH, CH)], gsem)

                @pl.when(g > 0)
                def _():
                    pltpu.make_async_copy(
                        table_ref.at[pl.ds(0, GCH)], rows3.at[o], gsem).wait()
                    pltpu.async_copy(
                        rows3.at[o],
                        out_refs[j].at[pl.ds(base + (g - 1) * GCH, GCH)], wsem)
                return carry

            lax.fori_loop(0, gr + 1, body, 0)
            pltpu.make_async_copy(table_ref.at[pl.ds(0, GCH)], rows3.at[0], wsem).wait()
            pltpu.make_async_copy(table_ref.at[pl.ds(0, GCH)], rows3.at[0], wsem).wait()

    res = k(table, *idxs)
    return list(res) if isinstance(res, (list, tuple)) else [res]


def _fill_zero(zero_s, rows):
    def zbody(i, carry):
        for j in range(HID // 16):
            zero_s[i, pl.ds(j * 16, 16)] = jnp.zeros((16,), _f32)
        return carry
    lax.fori_loop(0, rows, zbody, 0)


def _scatter_pipeline(rows_ref, idx_ref, acc, rows3, idx2, idxb,
                      rsem, isem, asem, base, gr, lo, bound, dump, ng):
    gch = ng * CH
    def body(g, carry):
        r = lax.rem(g, 2)
        o = 1 - r

        @pl.when(g < gr)
        def _():
            @pl.when(g >= 2)
            def _():
                pltpu.make_async_copy(
                    rows_ref.at[pl.ds(0, gch)], rows3.at[r], asem).wait()
            off = base + g * gch
            pltpu.async_copy(rows_ref.at[pl.ds(off, gch)], rows3.at[r], rsem)
            pltpu.async_copy(idx_ref.at[pl.ds(off, gch)], idx2.at[r], isem)

        @pl.when(g > 0)
        def _():
            pltpu.make_async_copy(rows_ref.at[pl.ds(0, gch)], rows3.at[o], rsem).wait()
            pltpu.make_async_copy(idx_ref.at[pl.ds(0, gch)], idx2.at[o], isem).wait()
            for bb in range(ng):
                row = o * ng + bb
                for j in range(CH // 16):
                    v = idx2[o, pl.ds(bb * CH + j * 16, 16)]
                    lv = v - lo
                    ok = (lv >= 0) & (lv < bound)
                    idxb[row, pl.ds(j * 16, 16)] = jnp.where(ok, lv, dump)
                pltpu.async_copy(rows3.at[o, pl.ds(bb * CH, CH)],
                                 acc.at[idxb.at[row]], asem, add=True)
        return carry

    lax.fori_loop(0, gr + 1, body, 0)
    pltpu.make_async_copy(rows_ref.at[pl.ds(0, gch)], rows3.at[0], asem).wait()
    pltpu.make_async_copy(rows_ref.at[pl.ds(0, gch)], rows3.at[0], asem).wait()


NGS = 3
GCHS = NGS * CH

_SC_SCR = [
    pltpu.VMEM((2, GCHS, HID), _f32),
    pltpu.VMEM((2, GCHS), jnp.int32),
    pltpu.VMEM((2 * NGS, CH), jnp.int32),
    pltpu.VMEM((128, HID), _f32),
    pltpu.SemaphoreType.DMA,
    pltpu.SemaphoreType.DMA,
    pltpu.SemaphoreType.DMA,
]


def _sc_scatter_n(rows, idx):
    per_w = EPAD // NW
    gr = per_w // GCHS
    tz = NACCROWS // NS
    wb = NPAD // NS

    @functools.partial(
        pl.kernel,
        out_type=jax.ShapeDtypeStruct((NC, NPAD, HID), _f32),
        mesh=_sc_mesh(),
        compiler_params=pltpu.CompilerParams(use_tc_tiling_on_sc=False),
        scratch_types=_SC_SCR + [pltpu.VMEM_SHARED((NACCROWS, HID), _f32)],
    )
    def k(rows_ref, idx_ref, out_ref, rows3, idx2, idxb, zero_s,
          rsem, isem, asem, acc):
        c = lax.axis_index("c")
        s = lax.axis_index("s")
        _fill_zero(zero_s, 128)
        zoff = s * tz
        for sz, o in ((128, 0), (128, 128), (128, 256), (128, 384), (tz - 512, 512)):
            pltpu.sync_copy(zero_s.at[pl.ds(0, sz)], acc.at[pl.ds(zoff + o, sz)])
        plsc.subcore_barrier()
        base = (s * NC + c) * per_w
        _scatter_pipeline(rows_ref, idx_ref, acc, rows3, idx2, idxb,
                          rsem, isem, asem, base, gr, 0, NPAD, NPAD, NGS)
        plsc.subcore_barrier()
        pltpu.sync_copy(acc.at[pl.ds(s * wb, wb)], out_ref.at[c, pl.ds(s * wb, wb)])

    return k(rows, idx)


def _sc_scatter_e(rows, idx_sorted, idx_loc8, zeros_z):
    per_tile = EPAD // NS
    ngrp = per_tile // GCHS
    tz = ACCROWS // NS
    wb = RACC // NS

    @functools.partial(
        pl.kernel,
        out_type=jax.ShapeDtypeStruct((EPAD, HID), _f32),
        mesh=_sc_mesh(),
        compiler_params=pltpu.CompilerParams(use_tc_tiling_on_sc=False,
                                             needs_layout_passes=False),
        scratch_types=[
            pltpu.VMEM((EPAD // NS,), jnp.int32),
            pltpu.VMEM((GCHS, HID), _f32),
            pltpu.VMEM((CH,), jnp.int32),
            pltpu.VMEM_SHARED((ACCROWS, HID), _f32),
        ],
    )
    def k(rows_ref, idx_ref, loc_ref, z_ref, out_ref, idx_all, rows_s, idxc, acc):
        c = lax.axis_index("c")
        s = lax.axis_index("s")
        base = s * per_tile
        pltpu.sync_copy(idx_ref.at[pl.ds(base, per_tile)], idx_all)
        for p in range(4):
            slot = NC * p + c
            lo = slot * RACC
            hi = lo + RACC
            pltpu.sync_copy(z_ref, acc.at[pl.ds(s * tz, tz)])
            plsc.subcore_barrier()

            def body(g, carry):
                goff = g * GCHS
                cmin = jnp.min(idx_all[pl.ds(goff, 16)], axis=0)
                cmax = jnp.max(idx_all[pl.ds(goff + GCHS - 16, 16)], axis=0)

                @pl.when((cmin < hi) & (cmax >= lo))
                def _():
                    pltpu.sync_copy(rows_ref.at[pl.ds(base + goff, GCHS)], rows_s)
                    for bb in range(NGS):
                        pltpu.sync_copy(
                            loc_ref.at[slot, pl.ds(base + goff + bb * CH, CH)], idxc)
                        pltpu.sync_copy(rows_s.at[pl.ds(bb * CH, CH)],
                                        acc.at[idxc], add=True)
                return carry

            lax.fori_loop(0, ngrp, body, 0)
            plsc.subcore_barrier()
            pltpu.sync_copy(acc.at[pl.ds(s * wb, wb)], out_ref.at[pl.ds(lo + s * wb, wb)])
            plsc.subcore_barrier()

    return k(rows, idx_sorted, idx_loc8, zeros_z)


def _k1_body(z_ref, wemb_ref, sc_ref, sh_ref, atoms_ref, scz_ref, shz_ref):
    z = z_ref[:, 0]
    oh = (jax.lax.broadcasted_iota(jnp.int32, (NB, 128), 1) == z[:, None]).astype(_f32)
    atoms_ref[...] = jnp.dot(oh, wemb_ref[...], preferred_element_type=_f32)
    scz_ref[...] = jnp.dot(oh, sc_ref[...], preferred_element_type=_f32)
    shz_ref[...] = jnp.dot(oh, sh_ref[...], preferred_element_type=_f32)


def _k1(z_pad, wemb_pad, scale_pad, shift_pad):
    return pl.pallas_call(
        _k1_body,
        grid=(NPAD // NB,),
        in_specs=[
            pl.BlockSpec((NB, 1), lambda i: (i, 0)),
            pl.BlockSpec((128, HID), lambda i: (0, 0)),
            pl.BlockSpec((128, 1), lambda i: (0, 0)),
            pl.BlockSpec((128, 1), lambda i: (0, 0)),
        ],
        out_specs=[
            pl.BlockSpec((NB, HID), lambda i: (i, 0)),
            pl.BlockSpec((NB, 1), lambda i: (i, 0)),
            pl.BlockSpec((NB, 1), lambda i: (i, 0)),
        ],
        out_shape=[
            jax.ShapeDtypeStruct((NPAD, HID), _f32),
            jax.ShapeDtypeStruct((NPAD, 1), _f32),
            jax.ShapeDtypeStruct((NPAD, 1), _f32),
        ],
    )(z_pad, wemb_pad, scale_pad, shift_pad)


def _k2_body(g0_ref, g1_ref, pbc_ref, cell_ref, wedge_ref, geom_ref, eattr_ref):
    off = jnp.dot(pbc_ref[...], cell_ref[...], preferred_element_type=_f32)
    d = g0_ref[...] - g1_ref[...] - off
    elen = jnp.sqrt(jnp.sum(d * d, axis=1))
    ez = _bessel(elen, CUT)
    c3 = _cutoff(elen, CUT3)
    geom_ref[...] = jnp.concatenate(
        [d[:, :3], elen[:, None], ez, c3[:, None], jnp.zeros((EB, 7), _f32)], axis=1)
    eattr_ref[...] = jnp.dot(ez, wedge_ref[...], preferred_element_type=_f32)


def _k2(g0, g1, pbc_pad, cell_pad, wedge):
    return pl.pallas_call(
        _k2_body,
        grid=(EPAD // EB,),
        in_specs=[
            pl.BlockSpec((EB, 16), lambda i: (i, 0)),
            pl.BlockSpec((EB, 16), lambda i: (i, 0)),
            pl.BlockSpec((EB, 16), lambda i: (i, 0)),
            pl.BlockSpec((16, 16), lambda i: (0, 0)),
            pl.BlockSpec((MAXN, HID), lambda i: (0, 0)),
        ],
        out_specs=[
            pl.BlockSpec((EB, 16), lambda i: (i, 0)),
            pl.BlockSpec((EB, HID), lambda i: (i, 0)),
        ],
        out_shape=[
            jax.ShapeDtypeStruct((EPAD, 16), _f32),
            jax.ShapeDtypeStruct((EPAD, HID), _f32),
        ],
    )(g0, g1, pbc_pad, cell_pad, wedge)


def _k3_body(gj_ref, gk_ref, out_ref):
    gj = gj_ref[...]
    gk = gk_ref[...]
    rij = gj[:, 3]
    rik = gk[:, 3]
    dot = jnp.sum(gj[:, :3] * gk[:, :3], axis=1)
    cos = dot / jnp.clip(rij * rik, 1e-8, None)
    cos = jnp.clip(cos, -1.0 + 1e-7, 1.0 - 1e-7)
    leg = jnp.stack(
        [jnp.ones_like(cos), cos, 0.5 * (3.0 * cos ** 2 - 1.0),
         0.5 * (5.0 * cos ** 3 - 3.0 * cos)], axis=-1)
    rad = gk[:, 4:8]
    three = (rad[:, None, :] * leg[:, :, None]).reshape(EB, MAXL * MAXN)
    w3 = gj[:, 8] * gk[:, 8]
    out_ref[...] = three * w3[:, None]


def _k3(gj, gk):
    return pl.pallas_call(
        _k3_body,
        grid=(EPAD // EB,),
        in_specs=[
            pl.BlockSpec((EB, 16), lambda i: (i, 0)),
            pl.BlockSpec((EB, 16), lambda i: (i, 0)),
        ],
        out_specs=pl.BlockSpec((EB, 16), lambda i: (i, 0)),
        out_shape=jax.ShapeDtypeStruct((EPAD, 16), _f32),
    )(gj, gk)


def _k4_body(tw_ref, ak_ref, wsbf_ref, wk_ref, m3_ref):
    a = jnp.dot(tw_ref[...], wsbf_ref[...], preferred_element_type=_f32)
    b = _sigmoid(jnp.dot(ak_ref[...], wk_ref[...], preferred_element_type=_f32))
    m3_ref[...] = a * b


def _k4(three_w, ak, wsbf_l, wk_l):
    return pl.pallas_call(
        _k4_body,
        grid=(EPAD // EB,),
        in_specs=[
            pl.BlockSpec((EB, 16), lambda i: (i, 0)),
            pl.BlockSpec((EB, HID), lambda i: (i, 0)),
            pl.BlockSpec((16, HID), lambda i: (0, 0)),
            pl.BlockSpec((HID, HID), lambda i: (0, 0)),
        ],
        out_specs=pl.BlockSpec((EB, HID), lambda i: (i, 0)),
        out_shape=jax.ShapeDtypeStruct((EPAD, HID), _f32),
    )(three_w, ak, wsbf_l, wk_l)


def _k6_body(a0_ref, a1_ref, eattr_ref, agg_ref, geom_ref,
             wg1_ref, wg2_ref, we_ref, wa_ref, wer_ref, war_ref,
             eout_ref, msg_ref):
    agg = agg_ref[...]
    eattr = eattr_ref[...]
    a0 = a0_ref[...]
    a1 = a1_ref[...]
    geom = geom_ref[...]

    g1 = jnp.dot(agg, wg1_ref[...], preferred_element_type=_f32)
    g2 = jnp.dot(agg, wg2_ref[...], preferred_element_type=_f32)
    eattr = eattr + _swish(g1) * _sigmoid(g2)

    er = jnp.dot(geom, wer_ref[...], preferred_element_type=_f32)
    ar = jnp.dot(geom, war_ref[...], preferred_element_type=_f32)

    a01 = jnp.concatenate([a0, a1], axis=1)
    we = we_ref[...]
    wa = wa_ref[...]
    p = jnp.dot(a01, we[:128], preferred_element_type=_f32)
    e1 = p[:, :HID] + jnp.dot(eattr, we[128:192, :HID], preferred_element_type=_f32)
    e2 = p[:, HID:] + jnp.dot(eattr, we[128:192, HID:], preferred_element_type=_f32)
    eattr = eattr + _swish(e1) * _sigmoid(e2) * er
    eout_ref[...] = eattr

    q = jnp.dot(a01, wa[:128], preferred_element_type=_f32)
    m1 = q[:, :HID] + jnp.dot(eattr, wa[128:192, :HID], preferred_element_type=_f32)
    m2 = q[:, HID:] + jnp.dot(eattr, wa[128:192, HID:], preferred_element_type=_f32)
    msg_ref[...] = _swish(m1) * _sigmoid(m2) * ar


def _k6(a0, a1, eattr, agg, geom, wg1_l, wg2_l, we_l, wa_l, wer_l, war_l):
    return pl.pallas_call(
        _k6_body,
        grid=(EPAD // EB,),
        in_specs=[
            pl.BlockSpec((EB, HID), lambda i: (i, 0)),
            pl.BlockSpec((EB, HID), lambda i: (i, 0)),
            pl.BlockSpec((EB, HID), lambda i: (i, 0)),
            pl.BlockSpec((EB, HID), lambda i: (i, 0)),
            pl.BlockSpec((EB, 16), lambda i: (i, 0)),
            pl.BlockSpec((HID, HID), lambda i: (0, 0)),
            pl.BlockSpec((HID, HID), lambda i: (0, 0)),
            pl.BlockSpec((192, 128), lambda i: (0, 0)),
            pl.BlockSpec((192, 128), lambda i: (0, 0)),
            pl.BlockSpec((16, HID), lambda i: (0, 0)),
            pl.BlockSpec((16, HID), lambda i: (0, 0)),
        ],
        out_specs=[
            pl.BlockSpec((EB, HID), lambda i: (i, 0)),
            pl.BlockSpec((EB, HID), lambda i: (i, 0)),
        ],
        out_shape=[
            jax.ShapeDtypeStruct((EPAD, HID), _f32),
            jax.ShapeDtypeStruct((EPAD, HID), _f32),
        ],
    )(a0, a1, eattr, agg, geom, wg1_l, wg2_l, we_l, wa_l, wer_l, war_l)


def _k8_body(a_ref, p_ref, out_ref):
    out_ref[...] = a_ref[...] + p_ref[0] + p_ref[1]


def _k8(atoms, part):
    return pl.pallas_call(
        _k8_body,
        grid=(NPAD // NB,),
        in_specs=[
            pl.BlockSpec((NB, HID), lambda i: (i, 0)),
            pl.BlockSpec((NC, NB, HID), lambda i: (0, i, 0)),
        ],
        out_specs=pl.BlockSpec((NB, HID), lambda i: (i, 0)),
        out_shape=jax.ShapeDtypeStruct((NPAD, HID), _f32),
    )(atoms, part)


def _k9_body(atoms_ref, scz_ref, shz_ref, f1_ref, f2_ref, f3_ref,
             v1_ref, v2_ref, v3_ref, out_ref):
    a = atoms_ref[...]
    h = _swish(jnp.dot(a, f1_ref[...], preferred_element_type=_f32))
    h = _swish(jnp.dot(h, f2_ref[...], preferred_element_type=_f32))
    h = jnp.dot(h, f3_ref[...], preferred_element_type=_f32)
    g = _swish(jnp.dot(a, v1_ref[...], preferred_element_type=_f32))
    g = _swish(jnp.dot(g, v2_ref[...], preferred_element_type=_f32))
    g = _sigmoid(jnp.dot(g, v3_ref[...], preferred_element_type=_f32))
    e = h[:, 0] * g[:, 0] * scz_ref[:, 0] + shz_ref[:, 0]

    @pl.when(pl.program_id(0) == 0)
    def _():
        out_ref[...] = jnp.zeros_like(out_ref)

    out_ref[...] += jnp.sum(e)[None, None]


def _k9(atoms, scz, shz, F1, F2, F3, V1, V2, V3):
    return pl.pallas_call(
        _k9_body,
        grid=(NPAD // NB,),
        in_specs=[
            pl.BlockSpec((NB, HID), lambda i: (i, 0)),
            pl.BlockSpec((NB, 1), lambda i: (i, 0)),
            pl.BlockSpec((NB, 1), lambda i: (i, 0)),
            pl.BlockSpec((HID, HID), lambda i: (0, 0)),
            pl.BlockSpec((HID, HID), lambda i: (0, 0)),
            pl.BlockSpec((HID, 1), lambda i: (0, 0)),
            pl.BlockSpec((HID, HID), lambda i: (0, 0)),
            pl.BlockSpec((HID, HID), lambda i: (0, 0)),
            pl.BlockSpec((HID, 1), lambda i: (0, 0)),
        ],
        out_specs=pl.BlockSpec((1, 1), lambda i: (0, 0)),
        out_shape=jax.ShapeDtypeStruct((1, 1), _f32),
    )(atoms, scz, shz, F1, F2, F3, V1, V2, V3)


def _pad1i(x, total, val):
    return jnp.full((total,), val, jnp.int32).at[:x.shape[0]].set(x.astype(jnp.int32))


def kernel(atom_pos, cell, pbc_offsets, atom_attr, edge_index,
           three_body_indices, num_three_body, num_triple_ij, num_atoms,
           num_bonds, num_graphs, W_embed, W_edge, Wsbf, Wk, Wg1, Wg2,
           We1, We2, Wer, Wa1, Wa2, War, F1, F2, F3, V1, V2, V3,
           scale, shift):
    i0 = edge_index[0]
    i1 = edge_index[1]
    t0 = three_body_indices[:, 0]
    t1 = three_body_indices[:, 1]

    i0g = _pad1i(i0, EPAD, 0)
    i1g = _pad1i(i1, EPAD, 0)
    i0s = _pad1i(i0, EPAD, NPAD)
    t0s = _pad1i(t0, EPAD, 2000000000)
    perm = jnp.argsort(t0s)
    t0_sorted = jnp.take(t0s, perm)
    slot_of = t0_sorted // RACC
    t0_loc8 = jnp.stack([
        jnp.where(slot_of == q, t0_sorted - q * RACC, RACC).astype(jnp.int32)
        for q in range(8)], axis=0)
    zeros_z = jnp.zeros((ACCROWS // NS, HID), _f32)
    t0g = jnp.take(_pad1i(t0, EPAD, 0), perm)
    t1g = jnp.take(_pad1i(t1, EPAD, 0), perm)

    z_pad = jnp.full((NPAD, 1), 127, jnp.int32).at[:N, 0].set(atom_attr[:, 0].astype(jnp.int32))
    wemb_pad = jnp.zeros((128, HID), _f32).at[:95].set(W_embed)
    scale_pad = jnp.zeros((128, 1), _f32).at[:95, 0].set(scale)
    shift_pad = jnp.zeros((128, 1), _f32).at[:95, 0].set(shift)
    pos_pad = jnp.zeros((N, 16), _f32).at[:, :3].set(atom_pos)
    pbc_pad = jnp.zeros((EPAD, 16), _f32).at[:E, :3].set(pbc_offsets)
    cell_pad = jnp.zeros((16, 16), _f32).at[:3, :3].set(cell[0])
    we_pack = jnp.concatenate([We1, We2], axis=2)
    wa_pack = jnp.concatenate([Wa1, Wa2], axis=2)
    wer_pad = jnp.zeros((NLAYERS, 16, HID), _f32).at[:, 4:8].set(Wer)
    war_pad = jnp.zeros((NLAYERS, 16, HID), _f32).at[:, 4:8].set(War)

    atoms, scz, shz = _k1(z_pad, wemb_pad, scale_pad, shift_pad)

    g0, g1 = _sc_gather(pos_pad, [i0g, i1g], 16)
    geom, eattr = _k2(g0, g1, pbc_pad, cell_pad, W_edge)

    gj, gk = _sc_gather(geom, [t0g, t1g], 16)
    three_w = _k3(gj, gk)

    for l in range(NLAYERS):
        a0, a1 = _sc_gather(atoms, [i0g, i1g], HID)
        (ak,) = _sc_gather(a1, [t1g], HID)
        m3 = _k4(three_w, ak, Wsbf[l], Wk[l])
        agg = _sc_scatter_e(m3, t0_sorted, t0_loc8, zeros_z)
        eattr, msg = _k6(a0, a1, eattr, agg, geom,
                         Wg1[l], Wg2[l], we_pack[l], wa_pack[l],
                         wer_pad[l], war_pad[l])
        part = _sc_scatter_n(msg, i0s)
        atoms = _k8(atoms, part)

    out = _k9(atoms, scz, shz, F1, F2, F3, V1, V2, V3)
    return out[0]
